# Initial kernel scaffold; baseline (speedup 1.0000x reference)
import jax
import jax.numpy as jnp
from jax.experimental import pallas as pl


def kernel(Wh_real, Wh_imag, W_real, W_imag, b_real, b_imag, N_neg, k_neighbors):
    raise NotImplementedError("write your pallas kernel here")


# speedup vs baseline: 319.1498x; 319.1498x over previous
import functools

import jax
import jax.numpy as jnp
from jax import lax
from jax.experimental import pallas as pl
from jax.experimental.pallas import tpu as pltpu
from jax.experimental.pallas import tpu_sc as plsc

_ROW_BLK = 2000
_CHUNK = 256
_LANES = 16
_NWORKERS = 32


def _proj_body(wr_ref, wi_ref, pr_ref, pi_ref, br_ref, bi_ref, out_ref):
    wr = wr_ref[...]
    wi = wi_ref[...]
    p_r = pr_ref[...]
    p_i = pi_ref[...]
    tr = jnp.sum(wr * p_r - wi * p_i, axis=1, keepdims=True) + br_ref[0, 0]
    ti = jnp.sum(wr * p_i + wi * p_r, axis=1, keepdims=True) + bi_ref[0, 0]
    rb = lax.bitcast_convert_type(tr, jnp.int32) + jnp.int32(0x8000)
    ib = lax.bitcast_convert_type(ti, jnp.int32) + jnp.int32(0x8000)
    word = jnp.bitwise_or(
        jnp.bitwise_and(ib, jnp.int32(-65536)),
        lax.shift_right_logical(rb, 16),
    )
    out_ref[...] = word


def _project_pack(Wh_real, Wh_imag, W_real, W_imag, b_real, b_imag):
    m, d = Wh_real.shape
    grid = m // _ROW_BLK
    out = pl.pallas_call(
        _proj_body,
        grid=(grid,),
        in_specs=[
            pl.BlockSpec((_ROW_BLK, d), lambda i: (i, 0)),
            pl.BlockSpec((_ROW_BLK, d), lambda i: (i, 0)),
            pl.BlockSpec((1, d), lambda i: (0, 0)),
            pl.BlockSpec((1, d), lambda i: (0, 0)),
            pl.BlockSpec((1, 1), lambda i: (0, 0)),
            pl.BlockSpec((1, 1), lambda i: (0, 0)),
        ],
        out_specs=pl.BlockSpec((_ROW_BLK, 1), lambda i: (i, 0)),
        out_shape=jax.ShapeDtypeStruct((m, 1), jnp.int32),
    )(Wh_real, Wh_imag, W_real, W_imag,
      b_real.reshape(1, 1), b_imag.reshape(1, 1))
    return out.reshape(m)


def _unpack_ri(word):
    r = plsc.bitcast(lax.shift_left(word, 16), jnp.float32)
    i = plsc.bitcast(jnp.bitwise_and(word, jnp.int32(-65536)), jnp.float32)
    return r, i


@functools.lru_cache(maxsize=None)
def _make_att_kernel(m, k, n):
    n_full = n // _CHUNK
    tail = n - n_full * _CHUNK
    n_chunks = n_full + (1 if tail else 0)
    full_groups = _CHUNK // _LANES
    tail_groups = tail // _LANES
    mesh = plsc.VectorSubcoreMesh(core_axis_name="c", subcore_axis_name="s")

    @functools.partial(
        pl.kernel,
        out_type=jax.ShapeDtypeStruct((k, n), jnp.float32),
        mesh=mesh,
        scratch_types=[
            pltpu.VMEM((m,), jnp.int32),
            pltpu.VMEM((k + 1, _CHUNK), jnp.int32),
            pltpu.VMEM((k, _CHUNK), jnp.float32),
        ],
        compiler_params=pltpu.CompilerParams(
            use_tc_tiling_on_sc=False, needs_layout_passes=False),
    )
    def att(tbl_hbm, nneg_hbm, out_hbm, tbl_v, idx_v, out_v):
        wid = lax.axis_index("s") * 2 + lax.axis_index("c")
        pltpu.sync_copy(tbl_hbm, tbl_v)
        my_chunks = (n_chunks - wid + _NWORKERS - 1) // _NWORKERS

        def chunk_body(t, carry):
            cid = wid + t * _NWORKERS
            col = pl.multiple_of(cid * _CHUNK, 128)
            is_tail = cid == n_full if tail else False

            @pl.when(jnp.logical_not(is_tail))
            def _():
                pltpu.sync_copy(nneg_hbm.at[:, pl.ds(col, _CHUNK)], idx_v)

            if tail:
                @pl.when(is_tail)
                def _():
                    pltpu.sync_copy(nneg_hbm.at[:, pl.ds(col, tail)],
                                    idx_v.at[:, pl.ds(0, tail)])

            def group_body(g, carry2):
                sl = pl.ds(g * _LANES, _LANES)
                cw = plsc.load_gather(tbl_v, [idx_v[0, sl]])
                cr, ci = _unpack_ri(cw)
                acc = jnp.full((_LANES,), 0.001, jnp.float32)
                atts = []
                for kk in range(k):
                    w = plsc.load_gather(tbl_v, [idx_v[kk + 1, sl]])
                    r, im = _unpack_ri(w)
                    a = jnp.maximum(cr * r + ci * im, 0.0)
                    acc = acc + a
                    atts.append(a)
                inv = 1.0 / acc
                for kk in range(k):
                    out_v[kk, sl] = atts[kk] * inv
                return carry2

            groups = jnp.where(is_tail, tail_groups, full_groups)
            lax.fori_loop(0, groups, group_body, 0)

            @pl.when(jnp.logical_not(is_tail))
            def _():
                pltpu.sync_copy(out_v, out_hbm.at[:, pl.ds(col, _CHUNK)])

            if tail:
                @pl.when(is_tail)
                def _():
                    pltpu.sync_copy(out_v.at[:, pl.ds(0, tail)],
                                    out_hbm.at[:, pl.ds(col, tail)])
            return carry

        lax.fori_loop(0, my_chunks, chunk_body, 0)

    return att


def kernel(Wh_real, Wh_imag, W_real, W_imag, b_real, b_imag, N_neg, k_neighbors):
    m, _ = Wh_real.shape
    kp1, n = N_neg.shape
    tbl = _project_pack(Wh_real, Wh_imag, W_real, W_imag, b_real, b_imag)
    att = _make_att_kernel(m, kp1 - 1, n)
    return att(tbl, N_neg)
